# Initial kernel scaffold; baseline (speedup 1.0000x reference)
import jax
import jax.numpy as jnp
from jax.experimental import pallas as pl


def kernel(x, codebook):
    raise NotImplementedError("write your pallas kernel here")


# speedup vs baseline: 1.0115x; 1.0115x over previous
import jax
import jax.numpy as jnp
from jax.experimental import pallas as pl

_BLOCK = 512


def _vq_kernel(x_ref, cb_ref, q_ref, idx_ref):
    x = x_ref[...]
    cb = cb_ref[...]
    x2 = jnp.sum(x * x, axis=-1, keepdims=True)
    c2 = jnp.sum(cb * cb, axis=-1)
    xc = jax.lax.dot_general(
        x, cb, (((1,), (1,)), ((), ())),
        preferred_element_type=jnp.float32)
    d2 = jnp.maximum(x2 + c2[None, :] - 2.0 * xc, 0.0)
    idx = jnp.argmin(d2, axis=-1).astype(jnp.int32)
    onehot = (jax.lax.broadcasted_iota(jnp.int32, d2.shape, 1)
              == idx[:, None]).astype(jnp.float32)
    q = jax.lax.dot_general(
        onehot, cb, (((1,), (0,)), ((), ())),
        preferred_element_type=jnp.float32,
        precision=jax.lax.Precision.HIGHEST)
    q_ref[...] = q
    idx_ref[...] = idx.reshape(1, 1, idx.shape[0])


def kernel(x, codebook):
    n, d = x.shape
    k = codebook.shape[0]
    grid = n // _BLOCK
    q, idx3 = pl.pallas_call(
        _vq_kernel,
        grid=(grid,),
        in_specs=[
            pl.BlockSpec((_BLOCK, d), lambda i: (i, 0)),
            pl.BlockSpec((k, d), lambda i: (0, 0)),
        ],
        out_specs=[
            pl.BlockSpec((_BLOCK, d), lambda i: (i, 0)),
            pl.BlockSpec((1, 1, _BLOCK), lambda i: (i, 0, 0)),
        ],
        out_shape=[
            jax.ShapeDtypeStruct((n, d), jnp.float32),
            jax.ShapeDtypeStruct((grid, 1, _BLOCK), jnp.int32),
        ],
    )(x, codebook)
    return q, idx3.reshape(n)


# speedup vs baseline: 1.6135x; 1.5952x over previous
import jax
import jax.numpy as jnp
from jax.experimental import pallas as pl
from jax.experimental.pallas import tpu as pltpu

_BLOCK = 512


def _vq_kernel(x_ref, cb_ref, q_ref, idx_ref, c2_ref, cbh_ref):
    i = pl.program_id(0)

    @pl.when(i == 0)
    def _():
        cbv = cb_ref[...]
        c2_ref[...] = jnp.sum(cbv * cbv, axis=-1)[None, :]
        cbh_ref[...] = cbv.astype(jnp.bfloat16)

    x = x_ref[...]
    x2 = jnp.sum(x * x, axis=-1, keepdims=True)
    xc = jax.lax.dot_general(
        x, cb_ref[...], (((1,), (1,)), ((), ())),
        preferred_element_type=jnp.float32)
    d2 = x2 + c2_ref[...] - 2.0 * xc
    k = d2.shape[1]
    m = jnp.min(d2, axis=-1, keepdims=True)
    iota = jax.lax.broadcasted_iota(jnp.int32, d2.shape, 1)
    idx = jnp.min(jnp.where(d2 == m, iota, k), axis=-1)
    onehot = jnp.where(iota == idx[:, None], 1.0, 0.0
                       ).astype(jnp.bfloat16)
    q = jax.lax.dot_general(
        onehot, cbh_ref[...], (((1,), (0,)), ((), ())),
        preferred_element_type=jnp.float32)
    q_ref[...] = q
    idx_ref[...] = idx.astype(jnp.int32).reshape(1, 1, idx.shape[0])


def kernel(x, codebook):
    n, d = x.shape
    k = codebook.shape[0]
    grid = n // _BLOCK
    q, idx3 = pl.pallas_call(
        _vq_kernel,
        grid=(grid,),
        in_specs=[
            pl.BlockSpec((_BLOCK, d), lambda i: (i, 0)),
            pl.BlockSpec((k, d), lambda i: (0, 0)),
        ],
        out_specs=[
            pl.BlockSpec((_BLOCK, d), lambda i: (i, 0)),
            pl.BlockSpec((1, 1, _BLOCK), lambda i: (i, 0, 0)),
        ],
        out_shape=[
            jax.ShapeDtypeStruct((n, d), jnp.float32),
            jax.ShapeDtypeStruct((grid, 1, _BLOCK), jnp.int32),
        ],
        scratch_shapes=[
            pltpu.VMEM((1, k), jnp.float32),
            pltpu.VMEM((k, d), jnp.bfloat16),
        ],
    )(x, codebook)
    return q, idx3.reshape(n)
